# Initial kernel scaffold; baseline (speedup 1.0000x reference)
import functools

import jax
import jax.numpy as jnp
from jax import lax
from jax.experimental import pallas as pl
from jax.experimental.pallas import tpu as pltpu
from jax.experimental.pallas import tpu_sc as plsc

N_NODES = 10000
D_FEAT = 128
NUM_CORES = 2
NUM_SUBCORES = 16
NUM_WORKERS = NUM_CORES * NUM_SUBCORES
CHUNK = 128
ACC_ROWS = 10240
ROWS_PER_TILE_ZERO = ACC_ROWS // NUM_SUBCORES
ROWS_PER_TILE_OUT = N_NODES // NUM_SUBCORES


def _sc_segment_sum(x, src, dst, nchunk):
    mesh = plsc.VectorSubcoreMesh(core_axis_name="c", subcore_axis_name="s")

    @functools.partial(
        pl.kernel,
        out_type=jax.ShapeDtypeStruct((NUM_CORES * N_NODES, D_FEAT), jnp.float32),
        mesh=mesh,
        scratch_types=[
            pltpu.VMEM((nchunk, CHUNK), jnp.int32),
            pltpu.VMEM((nchunk, CHUNK), jnp.int32),
            pltpu.VMEM((CHUNK, D_FEAT), jnp.float32),
            pltpu.VMEM_SHARED((ACC_ROWS, D_FEAT), jnp.float32),
            pltpu.SemaphoreType.DMA,
        ],
    )
    def kern(x_hbm, src_hbm, dst_hbm, out_hbm, sidx, didx, rows, acc, sem):
        cid = lax.axis_index("c")
        sid = lax.axis_index("s")
        wid = cid * NUM_SUBCORES + sid

        zeros = jnp.zeros((16,), jnp.float32)

        def zrow(r, _):
            for c in range(D_FEAT // 16):
                rows[r, pl.ds(c * 16, 16)] = zeros
            return 0

        lax.fori_loop(0, CHUNK, zrow, 0, unroll=False)
        for k in range(ROWS_PER_TILE_ZERO // CHUNK):
            pltpu.sync_copy(rows, acc.at[pl.ds(sid * ROWS_PER_TILE_ZERO + k * CHUNK, CHUNK)])

        pltpu.sync_copy(src_hbm.at[wid], sidx)
        pltpu.sync_copy(dst_hbm.at[wid], didx)

        plsc.subcore_barrier()

        def body(j, _):
            pltpu.async_copy(x_hbm.at[sidx.at[j]], rows, sem).wait()
            pltpu.sync_copy(rows, acc.at[didx.at[j]], add=True)
            return 0

        lax.fori_loop(0, nchunk, body, 0, unroll=False)

        plsc.subcore_barrier()

        pltpu.sync_copy(
            acc.at[pl.ds(sid * ROWS_PER_TILE_OUT, ROWS_PER_TILE_OUT)],
            out_hbm.at[pl.ds(cid * N_NODES + sid * ROWS_PER_TILE_OUT, ROWS_PER_TILE_OUT)],
        )

    return kern(x, src, dst)


def _combine_body(p0_ref, p1_ref, w_ref, o_ref):
    o_ref[...] = (p0_ref[...] + p1_ref[...]) * w_ref[...]


def _combine(partials, w, block=1000):
    nblk = N_NODES // block
    return pl.pallas_call(
        _combine_body,
        grid=(nblk,),
        in_specs=[
            pl.BlockSpec((block, D_FEAT), lambda i: (i, 0)),
            pl.BlockSpec((block, D_FEAT), lambda i: (i + N_NODES // block, 0)),
            pl.BlockSpec((1, D_FEAT), lambda i: (0, 0)),
        ],
        out_specs=pl.BlockSpec((block, D_FEAT), lambda i: (i, 0)),
        out_shape=jax.ShapeDtypeStruct((N_NODES, D_FEAT), jnp.float32),
    )(partials, partials, w)


def kernel(feature_matrix, edge_index, w):
    n_edges = edge_index.shape[1]
    src = edge_index[0].astype(jnp.int32)
    dst = edge_index[1].astype(jnp.int32)

    per_worker = -(-n_edges // NUM_WORKERS)
    nchunk = -(-per_worker // CHUNK)
    padded = NUM_WORKERS * nchunk * CHUNK
    pad = padded - n_edges
    src_p = jnp.concatenate([src, jnp.zeros((pad,), jnp.int32)]).reshape(
        NUM_WORKERS, nchunk, CHUNK)
    dst_p = jnp.concatenate([dst, jnp.full((pad,), N_NODES, jnp.int32)]).reshape(
        NUM_WORKERS, nchunk, CHUNK)

    partials = _sc_segment_sum(feature_matrix, src_p, dst_p, nchunk)
    return _combine(partials, w)


# speedup vs baseline: 4.9412x; 4.9412x over previous
import functools

import jax
import jax.numpy as jnp
from jax import lax
from jax.experimental import pallas as pl
from jax.experimental.pallas import tpu as pltpu
from jax.experimental.pallas import tpu_sc as plsc

N_NODES = 10000
D_FEAT = 128
NUM_CORES = 2
NUM_SUBCORES = 16
NUM_WORKERS = NUM_CORES * NUM_SUBCORES
CHUNK = 128
ACC_ROWS = 10240
ROWS_PER_TILE = ACC_ROWS // NUM_SUBCORES


def _sc_segment_sum(x, src, dst, nchunk):
    mesh = plsc.VectorSubcoreMesh(core_axis_name="c", subcore_axis_name="s")

    @functools.partial(
        pl.kernel,
        out_type=jax.ShapeDtypeStruct((NUM_CORES, ACC_ROWS, D_FEAT), jnp.float32),
        mesh=mesh,
        scratch_types=[
            pltpu.VMEM((nchunk, CHUNK), jnp.int32),
            pltpu.VMEM((nchunk, CHUNK), jnp.int32),
            pltpu.VMEM((CHUNK, D_FEAT), jnp.float32),
            pltpu.VMEM_SHARED((ACC_ROWS, D_FEAT), jnp.float32),
            pltpu.SemaphoreType.DMA,
        ],
    )
    def kern(x_hbm, src_hbm, dst_hbm, out_hbm, sidx, didx, rows, acc, sem):
        cid = lax.axis_index("c")
        sid = lax.axis_index("s")
        wid = cid * NUM_SUBCORES + sid

        zeros = jnp.zeros((16,), jnp.float32)

        def zrow(r, _):
            for c in range(D_FEAT // 16):
                rows[r, pl.ds(c * 16, 16)] = zeros
            return 0

        lax.fori_loop(0, CHUNK, zrow, 0, unroll=False)
        for k in range(ROWS_PER_TILE // CHUNK):
            pltpu.sync_copy(rows, acc.at[pl.ds(sid * ROWS_PER_TILE + k * CHUNK, CHUNK)])

        pltpu.sync_copy(src_hbm.at[wid], sidx)
        pltpu.sync_copy(dst_hbm.at[wid], didx)

        plsc.subcore_barrier()

        def body(j, _):
            pltpu.async_copy(x_hbm.at[sidx.at[j]], rows, sem).wait()
            pltpu.sync_copy(rows, acc.at[didx.at[j]], add=True)
            return 0

        lax.fori_loop(0, nchunk, body, 0, unroll=False)

        plsc.subcore_barrier()

        pltpu.sync_copy(
            acc.at[pl.ds(sid * ROWS_PER_TILE, ROWS_PER_TILE)],
            out_hbm.at[cid, pl.ds(sid * ROWS_PER_TILE, ROWS_PER_TILE)],
        )

    return kern(x, src, dst)


def _combine_body(p0_ref, p1_ref, w_ref, o_ref):
    o_ref[...] = (p0_ref[0] + p1_ref[0]) * w_ref[...]


def _combine(partials, w, block=1000):
    nblk = N_NODES // block
    return pl.pallas_call(
        _combine_body,
        grid=(nblk,),
        in_specs=[
            pl.BlockSpec((1, block, D_FEAT), lambda i: (0, i, 0)),
            pl.BlockSpec((1, block, D_FEAT), lambda i: (1, i, 0)),
            pl.BlockSpec((1, D_FEAT), lambda i: (0, 0)),
        ],
        out_specs=pl.BlockSpec((block, D_FEAT), lambda i: (i, 0)),
        out_shape=jax.ShapeDtypeStruct((N_NODES, D_FEAT), jnp.float32),
    )(partials, partials, w)


def kernel(feature_matrix, edge_index, w):
    n_edges = edge_index.shape[1]
    src = edge_index[0].astype(jnp.int32)
    dst = edge_index[1].astype(jnp.int32)

    per_worker = -(-n_edges // NUM_WORKERS)
    nchunk = -(-per_worker // CHUNK)
    padded = NUM_WORKERS * nchunk * CHUNK
    pad = padded - n_edges
    src_p = jnp.concatenate([src, jnp.zeros((pad,), jnp.int32)]).reshape(
        NUM_WORKERS, nchunk, CHUNK)
    dst_p = jnp.concatenate([dst, jnp.full((pad,), N_NODES, jnp.int32)]).reshape(
        NUM_WORKERS, nchunk, CHUNK)

    partials = _sc_segment_sum(feature_matrix, src_p, dst_p, nchunk)
    return _combine(partials, w)
